# baseline (device time: 149520 ns/iter reference)
import jax
import jax.numpy as jnp
from jax import lax
from jax.experimental import pallas as pl
from jax.experimental.pallas import tpu as pltpu

N_DEV = 16
M = 1024
N = 1024
CHUNK = M // N_DEV


def kernel(A, B):
    def body(a_ref, b_ref, out_ref, z_ref, comm_ref,
             rs_send, rs_recv, ag_send, ag_recv):
        my = lax.axis_index("i")
        left = lax.rem(my - 1 + N_DEV, N_DEV)
        right = lax.rem(my + 1, N_DEV)

        barrier = pltpu.get_barrier_semaphore()
        for nbr in (left, right):
            pl.semaphore_signal(barrier, inc=1, device_id=(nbr,),
                                device_id_type=pl.DeviceIdType.MESH)
        pl.semaphore_wait(barrier, 2)

        z_ref[...] = jnp.dot(a_ref[...], b_ref[...],
                             preferred_element_type=jnp.float32)

        for s in range(N_DEV - 1):
            send_c = lax.rem(my - s + N_DEV, N_DEV)
            recv_c = lax.rem(my - s - 1 + N_DEV, N_DEV)
            rdma = pltpu.make_async_remote_copy(
                src_ref=z_ref.at[pl.ds(send_c * CHUNK, CHUNK), :],
                dst_ref=comm_ref.at[s],
                send_sem=rs_send.at[s],
                recv_sem=rs_recv.at[s],
                device_id=(right,),
                device_id_type=pl.DeviceIdType.MESH,
            )
            rdma.start()
            rdma.wait()
            z_ref[pl.ds(recv_c * CHUNK, CHUNK), :] += comm_ref[s]

        own = lax.rem(my + 1, N_DEV)
        zc = z_ref[pl.ds(own * CHUNK, CHUNK), :]
        out_ref[pl.ds(own * CHUNK, CHUNK), :] = zc * (1.0 / (1.0 + jnp.exp(-zc)))

        for s in range(N_DEV - 1):
            send_c = lax.rem(my + 1 - s + N_DEV, N_DEV)
            rdma = pltpu.make_async_remote_copy(
                src_ref=out_ref.at[pl.ds(send_c * CHUNK, CHUNK), :],
                dst_ref=out_ref.at[pl.ds(send_c * CHUNK, CHUNK), :],
                send_sem=ag_send.at[s],
                recv_sem=ag_recv.at[s],
                device_id=(right,),
                device_id_type=pl.DeviceIdType.MESH,
            )
            rdma.start()
            rdma.wait()

    return pl.pallas_call(
        body,
        out_shape=jax.ShapeDtypeStruct((M, N), jnp.float32),
        in_specs=[pl.BlockSpec(memory_space=pltpu.VMEM),
                  pl.BlockSpec(memory_space=pltpu.VMEM)],
        out_specs=pl.BlockSpec(memory_space=pltpu.VMEM),
        scratch_shapes=[
            pltpu.VMEM((M, N), jnp.float32),
            pltpu.VMEM((N_DEV - 1, CHUNK, N), jnp.float32),
            pltpu.SemaphoreType.DMA((N_DEV - 1,)),
            pltpu.SemaphoreType.DMA((N_DEV - 1,)),
            pltpu.SemaphoreType.DMA((N_DEV - 1,)),
            pltpu.SemaphoreType.DMA((N_DEV - 1,)),
        ],
        compiler_params=pltpu.CompilerParams(collective_id=0),
    )(A, B)


# device time: 84854 ns/iter; 1.7621x vs baseline; 1.7621x over previous
import jax
import jax.numpy as jnp
from jax import lax
from jax.experimental import pallas as pl
from jax.experimental.pallas import tpu as pltpu

N_DEV = 16
M = 1024
N = 1024


def kernel(A, B):
    def body(a_ref, b_ref, out_ref, z_ref, comm_p, comm_z, ssems, rsems):
        my = lax.axis_index("i")
        p = my // 4
        q = my % 4
        plane = p * 4
        ring = {
            "P": (q, plane + (q + 1) % 4, plane + (q + 3) % 4),
            "Z": (p, ((p + 1) % 4) * 4 + q, ((p + 3) % 4) * 4 + q),
        }

        barrier = pltpu.get_barrier_semaphore()
        for d in (ring["P"][1], ring["P"][2], ring["Z"][1], ring["Z"][2]):
            pl.semaphore_signal(barrier, inc=1, device_id=(d,),
                                device_id_type=pl.DeviceIdType.MESH)
        pl.semaphore_wait(barrier, 4)

        sid_ctr = [0]

        def next_sid2():
            s = sid_ctr[0]
            sid_ctr[0] += 2
            return s

        def rs_round(s, axis, base, h, comm, slot0, ref):
            x, right, left = ring[axis]
            sid0 = next_sid2()

            def issue():
                rc = (x + 4 - s) % 4
                lc = (x + 2 + s) % 4
                r = pltpu.make_async_remote_copy(
                    src_ref=ref.at[pl.ds(base(rc), h), :],
                    dst_ref=comm.at[slot0],
                    send_sem=ssems.at[sid0], recv_sem=rsems.at[sid0],
                    device_id=(right,), device_id_type=pl.DeviceIdType.MESH)
                l = pltpu.make_async_remote_copy(
                    src_ref=ref.at[pl.ds(base(lc) + h, h), :],
                    dst_ref=comm.at[slot0 + 1],
                    send_sem=ssems.at[sid0 + 1], recv_sem=rsems.at[sid0 + 1],
                    device_id=(left,), device_id_type=pl.DeviceIdType.MESH)
                r.start()
                l.start()
                return r, l

            def finish(rl):
                r, l = rl
                r.wait()
                l.wait()
                ar = (x + 3 - s) % 4
                al = (x + 3 + s) % 4
                ref[pl.ds(base(ar), h), :] += comm[slot0]
                ref[pl.ds(base(al) + h, h), :] += comm[slot0 + 1]

            return issue, finish

        def ag_round(s, axis, base, h, ref):
            x, right, left = ring[axis]
            sid0 = next_sid2()

            def issue():
                rc = (x + 5 - s) % 4
                lc = (x + 1 + s) % 4
                r = pltpu.make_async_remote_copy(
                    src_ref=ref.at[pl.ds(base(rc), h), :],
                    dst_ref=ref.at[pl.ds(base(rc), h), :],
                    send_sem=ssems.at[sid0], recv_sem=rsems.at[sid0],
                    device_id=(right,), device_id_type=pl.DeviceIdType.MESH)
                l = pltpu.make_async_remote_copy(
                    src_ref=ref.at[pl.ds(base(lc) + h, h), :],
                    dst_ref=ref.at[pl.ds(base(lc) + h, h), :],
                    send_sem=ssems.at[sid0 + 1], recv_sem=rsems.at[sid0 + 1],
                    device_id=(left,), device_id_type=pl.DeviceIdType.MESH)
                r.start()
                l.start()
                return r, l

            def finish(rl):
                rl[0].wait()
                rl[1].wait()

            return issue, finish

        def build_half(row0, rows, ax1, ax2, comm_big, comm_small,
                       slot_b0, slot_s0):
            chunk = rows // 4
            sub = chunk // 4
            x1 = ring[ax1][0]
            x2 = ring[ax2][0]

            def base1(c):
                return row0 + c * chunk

            qb = row0 + ((x1 + 1) % 4) * chunk

            def base2(c):
                return qb + c * sub

            rounds = []
            for s in range(3):
                rounds.append(rs_round(s, ax1, base1, chunk // 2,
                                       comm_big, slot_b0 + 2 * s, z_ref))
            for s in range(3):
                rounds.append(rs_round(s, ax2, base2, sub // 2,
                                       comm_small, slot_s0 + 2 * s, z_ref))
            for s in range(3):
                rounds.append(ag_round(s, ax2, base2, sub // 2, out_ref))
            for s in range(3):
                rounds.append(ag_round(s, ax1, base1, chunk // 2, out_ref))
            silu_rows = qb + ((x2 + 1) % 4) * sub
            return rounds, silu_rows, sub

        def mm_block(c, row0, rows):
            b = row0 + c * rows
            z_ref[pl.ds(b, rows), :] = jnp.dot(
                a_ref[pl.ds(b, rows), :], b_ref[...],
                preferred_element_type=jnp.float32)

        rounds, silu_rows, sub = build_half(0, M, "P", "Z", comm_p, comm_z, 0, 0)

        mm_block(q, 0, 256)
        mm_block((q + 2) % 4, 0, 256)
        st = rounds[0][0]()
        mm_block((q + 1) % 4, 0, 256)
        mm_block((q + 3) % 4, 0, 256)
        rounds[0][1](st)

        for i in range(1, 12):
            st = rounds[i][0]()
            rounds[i][1](st)
            if i == 5:
                zc = z_ref[pl.ds(silu_rows, sub), :]
                out_ref[pl.ds(silu_rows, sub), :] = (
                    zc * (1.0 / (1.0 + jnp.exp(-zc))))

    return pl.pallas_call(
        body,
        out_shape=jax.ShapeDtypeStruct((M, N), jnp.float32),
        in_specs=[pl.BlockSpec(memory_space=pltpu.VMEM),
                  pl.BlockSpec(memory_space=pltpu.VMEM)],
        out_specs=pl.BlockSpec(memory_space=pltpu.VMEM),
        scratch_shapes=[
            pltpu.VMEM((M, N), jnp.float32),
            pltpu.VMEM((6, 128, N), jnp.float32),
            pltpu.VMEM((6, 32, N), jnp.float32),
            pltpu.SemaphoreType.DMA((48,)),
            pltpu.SemaphoreType.DMA((48,)),
        ],
        compiler_params=pltpu.CompilerParams(collective_id=0),
    )(A, B)


# device time: 74629 ns/iter; 2.0035x vs baseline; 1.1370x over previous
import jax
import jax.numpy as jnp
from jax import lax
from jax.experimental import pallas as pl
from jax.experimental.pallas import tpu as pltpu

N_DEV = 16
M = 1024
N = 1024


def kernel(A, B):
    def body(a_ref, b_ref, out_ref, z_ref, comm_pa, comm_za, comm_zb,
             comm_pb, ssems, rsems):
        my = lax.axis_index("i")
        p = my // 4
        q = my % 4
        plane = p * 4
        ring = {
            "P": (q, plane + (q + 1) % 4, plane + (q + 3) % 4),
            "Z": (p, ((p + 1) % 4) * 4 + q, ((p + 3) % 4) * 4 + q),
        }

        barrier = pltpu.get_barrier_semaphore()
        for d in (ring["P"][1], ring["P"][2], ring["Z"][1], ring["Z"][2]):
            pl.semaphore_signal(barrier, inc=1, device_id=(d,),
                                device_id_type=pl.DeviceIdType.MESH)
        pl.semaphore_wait(barrier, 4)

        sid_ctr = [0]

        def next_sid2():
            s = sid_ctr[0]
            sid_ctr[0] += 2
            return s

        def rs_round(s, axis, base, h, comm, slot0, ref):
            x, right, left = ring[axis]
            sid0 = next_sid2()

            def issue():
                rc = (x + 4 - s) % 4
                lc = (x + 2 + s) % 4
                r = pltpu.make_async_remote_copy(
                    src_ref=ref.at[pl.ds(base(rc), h), :],
                    dst_ref=comm.at[slot0],
                    send_sem=ssems.at[sid0], recv_sem=rsems.at[sid0],
                    device_id=(right,), device_id_type=pl.DeviceIdType.MESH)
                l = pltpu.make_async_remote_copy(
                    src_ref=ref.at[pl.ds(base(lc) + h, h), :],
                    dst_ref=comm.at[slot0 + 1],
                    send_sem=ssems.at[sid0 + 1], recv_sem=rsems.at[sid0 + 1],
                    device_id=(left,), device_id_type=pl.DeviceIdType.MESH)
                r.start()
                l.start()
                return r, l

            def finish(rl):
                r, l = rl
                r.wait()
                l.wait()
                ar = (x + 3 - s) % 4
                al = (x + 3 + s) % 4
                ref[pl.ds(base(ar), h), :] += comm[slot0]
                ref[pl.ds(base(al) + h, h), :] += comm[slot0 + 1]

            return issue, finish

        def ag_round(s, axis, base, h, ref):
            x, right, left = ring[axis]
            sid0 = next_sid2()

            def issue():
                rc = (x + 5 - s) % 4
                lc = (x + 1 + s) % 4
                r = pltpu.make_async_remote_copy(
                    src_ref=ref.at[pl.ds(base(rc), h), :],
                    dst_ref=ref.at[pl.ds(base(rc), h), :],
                    send_sem=ssems.at[sid0], recv_sem=rsems.at[sid0],
                    device_id=(right,), device_id_type=pl.DeviceIdType.MESH)
                l = pltpu.make_async_remote_copy(
                    src_ref=ref.at[pl.ds(base(lc) + h, h), :],
                    dst_ref=ref.at[pl.ds(base(lc) + h, h), :],
                    send_sem=ssems.at[sid0 + 1], recv_sem=rsems.at[sid0 + 1],
                    device_id=(left,), device_id_type=pl.DeviceIdType.MESH)
                r.start()
                l.start()
                return r, l

            def finish(rl):
                rl[0].wait()
                rl[1].wait()

            return issue, finish

        def build_half(row0, rows, ax1, ax2, comm_big, comm_small,
                       slot_b0, slot_s0):
            chunk = rows // 4
            sub = chunk // 4
            x1 = ring[ax1][0]
            x2 = ring[ax2][0]

            def base1(c):
                return row0 + c * chunk

            qb = row0 + ((x1 + 1) % 4) * chunk

            def base2(c):
                return qb + c * sub

            rounds = []
            for s in range(3):
                rounds.append(rs_round(s, ax1, base1, chunk // 2,
                                       comm_big, slot_b0 + 2 * s, z_ref))
            for s in range(3):
                rounds.append(rs_round(s, ax2, base2, sub // 2,
                                       comm_small, slot_s0 + 2 * s, z_ref))
            for s in range(3):
                rounds.append(ag_round(s, ax2, base2, sub // 2, out_ref))
            for s in range(3):
                rounds.append(ag_round(s, ax1, base1, chunk // 2, out_ref))
            silu_rows = qb + ((x2 + 1) % 4) * sub
            return rounds, silu_rows, sub

        def mm_block(c, row0, rows):
            b = row0 + c * rows
            z_ref[pl.ds(b, rows), :] = jnp.dot(
                a_ref[pl.ds(b, rows), :], b_ref[...],
                preferred_element_type=jnp.float32)

        ra, silu_a, sub_a = build_half(0, 768, "P", "Z", comm_pa, comm_za, 0, 0)
        rb, silu_b, sub_b = build_half(768, 256, "Z", "P", comm_zb, comm_pb, 0, 0)

        mm_block(q, 0, 192)
        mm_block((q + 2) % 4, 0, 192)
        mm_block(p, 768, 64)
        mm_block((p + 2) % 4, 768, 64)
        sta = ra[0][0]()
        stb = rb[0][0]()
        mm_block((q + 1) % 4, 0, 192)
        mm_block((q + 3) % 4, 0, 192)
        mm_block((p + 1) % 4, 768, 64)
        mm_block((p + 3) % 4, 768, 64)
        ra[0][1](sta)
        rb[0][1](stb)

        for i in range(1, 12):
            sta = ra[i][0]()
            stb = rb[i][0]()
            ra[i][1](sta)
            rb[i][1](stb)
            if i == 5:
                for rows0, sb in ((silu_a, sub_a), (silu_b, sub_b)):
                    zc = z_ref[pl.ds(rows0, sb), :]
                    out_ref[pl.ds(rows0, sb), :] = (
                        zc * (1.0 / (1.0 + jnp.exp(-zc))))

    return pl.pallas_call(
        body,
        out_shape=jax.ShapeDtypeStruct((M, N), jnp.float32),
        in_specs=[pl.BlockSpec(memory_space=pltpu.VMEM),
                  pl.BlockSpec(memory_space=pltpu.VMEM)],
        out_specs=pl.BlockSpec(memory_space=pltpu.VMEM),
        scratch_shapes=[
            pltpu.VMEM((M, N), jnp.float32),
            pltpu.VMEM((6, 96, N), jnp.float32),
            pltpu.VMEM((6, 24, N), jnp.float32),
            pltpu.VMEM((6, 32, N), jnp.float32),
            pltpu.VMEM((6, 8, N), jnp.float32),
            pltpu.SemaphoreType.DMA((96,)),
            pltpu.SemaphoreType.DMA((96,)),
        ],
        compiler_params=pltpu.CompilerParams(collective_id=0),
    )(A, B)


# device time: 59023 ns/iter; 2.5332x vs baseline; 1.2644x over previous
import jax
import jax.numpy as jnp
from jax import lax
from jax.experimental import pallas as pl
from jax.experimental.pallas import tpu as pltpu

N_DEV = 16
M = 1024
N = 1024


def kernel(A, B):
    def body(a_ref, b_ref, out_ref, b16_ref, z_ref, comm_pa, comm_za,
             comm_zb, comm_pb, ssems, rsems):
        my = lax.axis_index("i")
        p = my // 4
        q = my % 4
        plane = p * 4
        ring = {
            "P": (q, plane + (q + 1) % 4, plane + (q + 3) % 4),
            "Z": (p, ((p + 1) % 4) * 4 + q, ((p + 3) % 4) * 4 + q),
        }

        barrier = pltpu.get_barrier_semaphore()
        for d in (ring["P"][1], ring["P"][2], ring["Z"][1], ring["Z"][2]):
            pl.semaphore_signal(barrier, inc=1, device_id=(d,),
                                device_id_type=pl.DeviceIdType.MESH)
        pl.semaphore_wait(barrier, 4)

        sid_ctr = [0]

        def next_sid2():
            s = sid_ctr[0]
            sid_ctr[0] += 2
            return s

        def rs_round(s, axis, base, h, comm, slot0, ref):
            x, right, left = ring[axis]
            sid0 = next_sid2()

            def issue():
                rc = (x + 4 - s) % 4
                lc = (x + 2 + s) % 4
                r = pltpu.make_async_remote_copy(
                    src_ref=ref.at[pl.ds(base(rc), h), :],
                    dst_ref=comm.at[slot0],
                    send_sem=ssems.at[sid0], recv_sem=rsems.at[sid0],
                    device_id=(right,), device_id_type=pl.DeviceIdType.MESH)
                l = pltpu.make_async_remote_copy(
                    src_ref=ref.at[pl.ds(base(lc) + h, h), :],
                    dst_ref=comm.at[slot0 + 1],
                    send_sem=ssems.at[sid0 + 1], recv_sem=rsems.at[sid0 + 1],
                    device_id=(left,), device_id_type=pl.DeviceIdType.MESH)
                r.start()
                l.start()
                return r, l

            def finish(rl):
                r, l = rl
                r.wait()
                l.wait()
                ar = (x + 3 - s) % 4
                al = (x + 3 + s) % 4
                ref[pl.ds(base(ar), h), :] += comm[slot0]
                ref[pl.ds(base(al) + h, h), :] += comm[slot0 + 1]

            return issue, finish

        def ag_round(s, axis, base, h, ref):
            x, right, left = ring[axis]
            sid0 = next_sid2()

            def issue():
                rc = (x + 5 - s) % 4
                lc = (x + 1 + s) % 4
                r = pltpu.make_async_remote_copy(
                    src_ref=ref.at[pl.ds(base(rc), h), :],
                    dst_ref=ref.at[pl.ds(base(rc), h), :],
                    send_sem=ssems.at[sid0], recv_sem=rsems.at[sid0],
                    device_id=(right,), device_id_type=pl.DeviceIdType.MESH)
                l = pltpu.make_async_remote_copy(
                    src_ref=ref.at[pl.ds(base(lc) + h, h), :],
                    dst_ref=ref.at[pl.ds(base(lc) + h, h), :],
                    send_sem=ssems.at[sid0 + 1], recv_sem=rsems.at[sid0 + 1],
                    device_id=(left,), device_id_type=pl.DeviceIdType.MESH)
                r.start()
                l.start()
                return r, l

            def finish(rl):
                rl[0].wait()
                rl[1].wait()

            return issue, finish

        def build_half(row0, rows, ax1, ax2, comm_big, comm_small,
                       slot_b0, slot_s0):
            chunk = rows // 4
            sub = chunk // 4
            x1 = ring[ax1][0]
            x2 = ring[ax2][0]

            def base1(c):
                return row0 + c * chunk

            qb = row0 + ((x1 + 1) % 4) * chunk

            def base2(c):
                return qb + c * sub

            rounds = []
            for s in range(3):
                rounds.append(rs_round(s, ax1, base1, chunk // 2,
                                       comm_big, slot_b0 + 2 * s, z_ref))
            for s in range(3):
                rounds.append(rs_round(s, ax2, base2, sub // 2,
                                       comm_small, slot_s0 + 2 * s, z_ref))
            for s in range(3):
                rounds.append(ag_round(s, ax2, base2, sub // 2, out_ref))
            for s in range(3):
                rounds.append(ag_round(s, ax1, base1, chunk // 2, out_ref))
            silu_rows = qb + ((x2 + 1) % 4) * sub
            return rounds, silu_rows, sub

        b16_ref[...] = b_ref[...].astype(jnp.bfloat16)

        def mm_block(c, row0, rows):
            b = row0 + c * rows
            z_ref[pl.ds(b, rows), :] = jnp.dot(
                a_ref[pl.ds(b, rows), :].astype(jnp.bfloat16), b16_ref[...],
                preferred_element_type=jnp.float32).astype(jnp.bfloat16)

        ra, silu_a, sub_a = build_half(0, 512, "P", "Z", comm_pa, comm_za, 0, 0)
        rb, silu_b, sub_b = build_half(512, 512, "Z", "P", comm_zb, comm_pb, 0, 0)

        mm_block(q, 0, 128)
        mm_block((q + 2) % 4, 0, 128)
        mm_block(p, 512, 128)
        mm_block((p + 2) % 4, 512, 128)
        sta = ra[0][0]()
        stb = rb[0][0]()
        mm_block((q + 1) % 4, 0, 128)
        mm_block((q + 3) % 4, 0, 128)
        mm_block((p + 1) % 4, 512, 128)
        mm_block((p + 3) % 4, 512, 128)
        ra[0][1](sta)
        rb[0][1](stb)

        for i in range(1, 12):
            sta = ra[i][0]()
            stb = rb[i][0]()
            ra[i][1](sta)
            rb[i][1](stb)
            if i == 5:
                for rows0, sb in ((silu_a, sub_a), (silu_b, sub_b)):
                    zc = z_ref[pl.ds(rows0, sb), :].astype(jnp.float32)
                    out_ref[pl.ds(rows0, sb), :] = (
                        zc * (1.0 / (1.0 + jnp.exp(-zc)))
                    ).astype(jnp.bfloat16)

    return pl.pallas_call(
        body,
        out_shape=jax.ShapeDtypeStruct((M, N), jnp.bfloat16),
        in_specs=[pl.BlockSpec(memory_space=pltpu.VMEM),
                  pl.BlockSpec(memory_space=pltpu.VMEM)],
        out_specs=pl.BlockSpec(memory_space=pltpu.VMEM),
        scratch_shapes=[
            pltpu.VMEM((512, N), jnp.bfloat16),
            pltpu.VMEM((M, N), jnp.bfloat16),
            pltpu.VMEM((6, 64, N), jnp.bfloat16),
            pltpu.VMEM((6, 16, N), jnp.bfloat16),
            pltpu.VMEM((6, 64, N), jnp.bfloat16),
            pltpu.VMEM((6, 16, N), jnp.bfloat16),
            pltpu.SemaphoreType.DMA((96,)),
            pltpu.SemaphoreType.DMA((96,)),
        ],
        compiler_params=pltpu.CompilerParams(collective_id=0),
    )(A, B)


# device time: 52330 ns/iter; 2.8573x vs baseline; 1.1279x over previous
import jax
import jax.numpy as jnp
from jax import lax
from jax.experimental import pallas as pl
from jax.experimental.pallas import tpu as pltpu

N_DEV = 16
M = 1024
N = 1024


def kernel(A, B):
    def body(a_ref, b_ref, out_ref, b16_ref, z_ref, comm_pa, comm_zb,
             m1a, m2a, m1b, m2b, ssems, rsems):
        my = lax.axis_index("i")
        p = my // 4
        q = my % 4
        plane = p * 4

        def lab2plane(l):
            return jnp.where(l == 0, 0, jnp.where(l == 1, 2,
                             jnp.where(l == 2, 3, 1)))

        pos_z = jnp.where(p == 0, 0, jnp.where(p == 1, 3,
                          jnp.where(p == 2, 1, 2)))
        ring = {
            "P": (q, plane + (q + 1) % 4, plane + (q + 3) % 4),
            "Z": (pos_z, lab2plane((pos_z + 1) % 4) * 4 + q,
                  lab2plane((pos_z + 3) % 4) * 4 + q),
        }
        qb0, qb1 = q % 2, q // 2
        pb0, pb1 = p % 2, p // 2
        bfly = {
            "P": (q, qb0, qb1, plane + q + 1 - 2 * qb0,
                  plane + (q + 2 - 4 * qb1)),
            "Z": (p, pb0, pb1, (p + 1 - 2 * pb0) * 4 + q,
                  (p + 2 - 4 * pb1) * 4 + q),
        }

        barrier = pltpu.get_barrier_semaphore()
        for d in (ring["P"][1], ring["P"][2], ring["Z"][1], ring["Z"][2],
                  bfly["P"][4]):
            pl.semaphore_signal(barrier, inc=1, device_id=(d,),
                                device_id_type=pl.DeviceIdType.MESH)
        pl.semaphore_wait(barrier, 5)

        sid_ctr = [0]

        def next_sid2():
            s = sid_ctr[0]
            sid_ctr[0] += 2
            return s

        def rs_round(s, axis, base, h, comm, slot0, ref):
            x, right, left = ring[axis]
            sid0 = next_sid2()

            def issue():
                rc = (x + 4 - s) % 4
                lc = (x + 2 + s) % 4
                r = pltpu.make_async_remote_copy(
                    src_ref=ref.at[pl.ds(base(rc), h), :],
                    dst_ref=comm.at[slot0],
                    send_sem=ssems.at[sid0], recv_sem=rsems.at[sid0],
                    device_id=(right,), device_id_type=pl.DeviceIdType.MESH)
                l = pltpu.make_async_remote_copy(
                    src_ref=ref.at[pl.ds(base(lc) + h, h), :],
                    dst_ref=comm.at[slot0 + 1],
                    send_sem=ssems.at[sid0 + 1], recv_sem=rsems.at[sid0 + 1],
                    device_id=(left,), device_id_type=pl.DeviceIdType.MESH)
                r.start()
                l.start()
                return r, l

            def finish(rl):
                r, l = rl
                r.wait()
                l.wait()
                ar = (x + 3 - s) % 4
                al = (x + 3 + s) % 4
                ref[pl.ds(base(ar), h), :] += comm[slot0]
                ref[pl.ds(base(al) + h, h), :] += comm[slot0 + 1]

            return issue, finish

        def ag_round(s, axis, base, h, ref):
            x, right, left = ring[axis]
            sid0 = next_sid2()

            def issue():
                rc = (x + 5 - s) % 4
                lc = (x + 1 + s) % 4
                r = pltpu.make_async_remote_copy(
                    src_ref=ref.at[pl.ds(base(rc), h), :],
                    dst_ref=ref.at[pl.ds(base(rc), h), :],
                    send_sem=ssems.at[sid0], recv_sem=rsems.at[sid0],
                    device_id=(right,), device_id_type=pl.DeviceIdType.MESH)
                l = pltpu.make_async_remote_copy(
                    src_ref=ref.at[pl.ds(base(lc) + h, h), :],
                    dst_ref=ref.at[pl.ds(base(lc) + h, h), :],
                    send_sem=ssems.at[sid0 + 1], recv_sem=rsems.at[sid0 + 1],
                    device_id=(left,), device_id_type=pl.DeviceIdType.MESH)
                r.start()
                l.start()
                return r, l

            def finish(rl):
                rl[0].wait()
                rl[1].wait()

            return issue, finish

        def bfly_rs_round(axis, dist, qb, comm):
            _, b0, b1, d1, d2 = bfly[axis]
            if dist == 1:
                partner, nrows = d1, 64
                keep = qb + b0 * 64
                send = qb + (1 - b0) * 64
            else:
                partner, nrows = d2, 32
                qbh = qb + b0 * 64
                keep = qbh + b1 * 32
                send = qbh + (1 - b1) * 32
            sid0 = next_sid2()

            def issue():
                r = pltpu.make_async_remote_copy(
                    src_ref=z_ref.at[pl.ds(send, nrows), :],
                    dst_ref=comm,
                    send_sem=ssems.at[sid0], recv_sem=rsems.at[sid0],
                    device_id=(partner,), device_id_type=pl.DeviceIdType.MESH)
                r.start()
                return (r,)

            def finish(st):
                st[0].wait()
                z_ref[pl.ds(keep, nrows), :] += comm[...]

            return issue, finish

        def bfly_ag_round(axis, dist, qb):
            _, b0, b1, d1, d2 = bfly[axis]
            if dist == 2:
                partner, nrows = d2, 32
                off = qb + b0 * 64 + b1 * 32
            else:
                partner, nrows = d1, 64
                off = qb + b0 * 64
            sid0 = next_sid2()

            def issue():
                r = pltpu.make_async_remote_copy(
                    src_ref=out_ref.at[pl.ds(off, nrows), :],
                    dst_ref=out_ref.at[pl.ds(off, nrows), :],
                    send_sem=ssems.at[sid0], recv_sem=rsems.at[sid0],
                    device_id=(partner,), device_id_type=pl.DeviceIdType.MESH)
                r.start()
                return (r,)

            def finish(st):
                st[0].wait()

            return issue, finish

        def build_half(row0, ax1, ax2, comm_big, slot_b0, comm_m1, comm_m2):
            chunk = 128
            x1 = ring[ax1][0]

            def base1(c):
                return row0 + c * chunk

            qb = row0 + ((x1 + 1) % 4) * chunk

            rounds = []
            for s in range(3):
                rounds.append(rs_round(s, ax1, base1, chunk // 2,
                                       comm_big, slot_b0 + 2 * s, z_ref))
            rounds.append(bfly_rs_round(ax2, 1, qb, comm_m1))
            rounds.append(bfly_rs_round(ax2, 2, qb, comm_m2))
            rounds.append(bfly_ag_round(ax2, 2, qb))
            rounds.append(bfly_ag_round(ax2, 1, qb))
            for s in range(3):
                rounds.append(ag_round(s, ax1, base1, chunk // 2, out_ref))
            _, b0, b1, _, _ = bfly[ax2]
            silu_rows = qb + b0 * 64 + b1 * 32
            return rounds, silu_rows, 32

        b16_ref[...] = b_ref[...].astype(jnp.bfloat16)

        def mm_block(c, row0, rows):
            b = row0 + c * rows
            z_ref[pl.ds(b, rows), :] = jnp.dot(
                a_ref[pl.ds(b, rows), :].astype(jnp.bfloat16), b16_ref[...],
                preferred_element_type=jnp.float32).astype(jnp.bfloat16)

        ra, silu_a, sub_a = build_half(0, "P", "Z", comm_pa, 0, m1a, m2a)
        rb, silu_b, sub_b = build_half(512, "Z", "P", comm_zb, 0, m1b, m2b)

        mm_block(q, 0, 128)
        mm_block((q + 2) % 4, 0, 128)
        mm_block(pos_z, 512, 128)
        mm_block((pos_z + 2) % 4, 512, 128)
        sta = ra[0][0]()
        stb = rb[0][0]()
        mm_block((q + 1) % 4, 0, 128)
        mm_block((q + 3) % 4, 0, 128)
        mm_block((pos_z + 1) % 4, 512, 128)
        mm_block((pos_z + 3) % 4, 512, 128)
        ra[0][1](sta)
        rb[0][1](stb)

        for i in range(1, 10):
            sta = ra[i][0]()
            stb = rb[i][0]()
            ra[i][1](sta)
            rb[i][1](stb)
            if i == 4:
                for rows0, sb in ((silu_a, sub_a), (silu_b, sub_b)):
                    zc = z_ref[pl.ds(rows0, sb), :].astype(jnp.float32)
                    out_ref[pl.ds(rows0, sb), :] = (
                        zc * (1.0 / (1.0 + jnp.exp(-zc)))
                    ).astype(jnp.bfloat16)

    return pl.pallas_call(
        body,
        out_shape=jax.ShapeDtypeStruct((M, N), jnp.bfloat16),
        in_specs=[pl.BlockSpec(memory_space=pltpu.VMEM),
                  pl.BlockSpec(memory_space=pltpu.VMEM)],
        out_specs=pl.BlockSpec(memory_space=pltpu.VMEM),
        scratch_shapes=[
            pltpu.VMEM((512, N), jnp.bfloat16),
            pltpu.VMEM((M, N), jnp.bfloat16),
            pltpu.VMEM((6, 64, N), jnp.bfloat16),
            pltpu.VMEM((6, 64, N), jnp.bfloat16),
            pltpu.VMEM((64, N), jnp.bfloat16),
            pltpu.VMEM((32, N), jnp.bfloat16),
            pltpu.VMEM((64, N), jnp.bfloat16),
            pltpu.VMEM((32, N), jnp.bfloat16),
            pltpu.SemaphoreType.DMA((64,)),
            pltpu.SemaphoreType.DMA((64,)),
        ],
        compiler_params=pltpu.CompilerParams(collective_id=0),
    )(A, B)


# device time: 51021 ns/iter; 2.9306x vs baseline; 1.0257x over previous
import jax
import jax.numpy as jnp
from jax import lax
from jax.experimental import pallas as pl
from jax.experimental.pallas import tpu as pltpu

N_DEV = 16
M = 1024
N = 1024


def kernel(A, B):
    def body(a_ref, b_ref, out_ref, b16_ref, z_ref, comm_pa, comm_zb,
             m1a, m2a, m1b, m2b, ssems, rsems):
        my = lax.axis_index("i")
        p = my // 4
        q = my % 4
        plane = p * 4

        def lab2plane(l):
            return jnp.where(l == 0, 0, jnp.where(l == 1, 2,
                             jnp.where(l == 2, 3, 1)))

        pos_z = jnp.where(p == 0, 0, jnp.where(p == 1, 3,
                          jnp.where(p == 2, 1, 2)))
        ring = {
            "P": (q, plane + (q + 1) % 4, plane + (q + 3) % 4),
            "Z": (pos_z, lab2plane((pos_z + 1) % 4) * 4 + q,
                  lab2plane((pos_z + 3) % 4) * 4 + q),
        }
        qb0, qb1 = q % 2, q // 2
        pb0, pb1 = p % 2, p // 2
        bfly = {
            "P": (q, qb0, qb1, plane + q + 1 - 2 * qb0,
                  plane + (q + 2 - 4 * qb1)),
            "Z": (p, pb0, pb1, (p + 1 - 2 * pb0) * 4 + q,
                  (p + 2 - 4 * pb1) * 4 + q),
        }

        barrier = pltpu.get_barrier_semaphore()
        for d in (ring["P"][1], ring["P"][2], ring["Z"][1], ring["Z"][2],
                  bfly["P"][4]):
            pl.semaphore_signal(barrier, inc=1, device_id=(d,),
                                device_id_type=pl.DeviceIdType.MESH)
        pl.semaphore_wait(barrier, 5)

        sid_ctr = [0]

        def next_sid2():
            s = sid_ctr[0]
            sid_ctr[0] += 2
            return s

        def rs_round(s, axis, base, h, comm, slot0, ref):
            x, right, left = ring[axis]
            sid0 = next_sid2()

            def issue():
                rc = (x + 4 - s) % 4
                lc = (x + 2 + s) % 4
                r = pltpu.make_async_remote_copy(
                    src_ref=ref.at[pl.ds(base(rc), h), :],
                    dst_ref=comm.at[slot0],
                    send_sem=ssems.at[sid0], recv_sem=rsems.at[sid0],
                    device_id=(right,), device_id_type=pl.DeviceIdType.MESH)
                l = pltpu.make_async_remote_copy(
                    src_ref=ref.at[pl.ds(base(lc) + h, h), :],
                    dst_ref=comm.at[slot0 + 1],
                    send_sem=ssems.at[sid0 + 1], recv_sem=rsems.at[sid0 + 1],
                    device_id=(left,), device_id_type=pl.DeviceIdType.MESH)
                r.start()
                l.start()
                return r, l

            def finish(rl):
                r, l = rl
                ar = (x + 3 - s) % 4
                al = (x + 3 + s) % 4
                r.wait()
                ref[pl.ds(base(ar), h), :] += comm[slot0]
                l.wait()
                ref[pl.ds(base(al) + h, h), :] += comm[slot0 + 1]

            return issue, finish

        def ag_round(s, axis, base, h, ref):
            x, right, left = ring[axis]
            sid0 = next_sid2()

            def issue():
                rc = (x + 5 - s) % 4
                lc = (x + 1 + s) % 4
                r = pltpu.make_async_remote_copy(
                    src_ref=ref.at[pl.ds(base(rc), h), :],
                    dst_ref=ref.at[pl.ds(base(rc), h), :],
                    send_sem=ssems.at[sid0], recv_sem=rsems.at[sid0],
                    device_id=(right,), device_id_type=pl.DeviceIdType.MESH)
                l = pltpu.make_async_remote_copy(
                    src_ref=ref.at[pl.ds(base(lc) + h, h), :],
                    dst_ref=ref.at[pl.ds(base(lc) + h, h), :],
                    send_sem=ssems.at[sid0 + 1], recv_sem=rsems.at[sid0 + 1],
                    device_id=(left,), device_id_type=pl.DeviceIdType.MESH)
                r.start()
                l.start()
                return r, l

            def finish(rl):
                rl[0].wait()
                rl[1].wait()

            return issue, finish

        def bfly_rs_round(axis, dist, qb, comm):
            _, b0, b1, d1, d2 = bfly[axis]
            if dist == 1:
                partner, nrows = d1, 64
                keep = qb + b0 * 64
                send = qb + (1 - b0) * 64
            else:
                partner, nrows = d2, 32
                qbh = qb + b0 * 64
                keep = qbh + b1 * 32
                send = qbh + (1 - b1) * 32
            sid0 = next_sid2()

            def issue():
                r = pltpu.make_async_remote_copy(
                    src_ref=z_ref.at[pl.ds(send, nrows), :],
                    dst_ref=comm,
                    send_sem=ssems.at[sid0], recv_sem=rsems.at[sid0],
                    device_id=(partner,), device_id_type=pl.DeviceIdType.MESH)
                r.start()
                return (r,)

            def finish(st):
                st[0].wait()
                z_ref[pl.ds(keep, nrows), :] += comm[...]

            return issue, finish

        def bfly_x2_round(axis, qb, comm):
            _, b0, b1, d1, d2 = bfly[axis]
            off = qb + b0 * 64
            sid0 = next_sid2()

            def issue():
                r = pltpu.make_async_remote_copy(
                    src_ref=z_ref.at[pl.ds(off, 64), :],
                    dst_ref=comm,
                    send_sem=ssems.at[sid0], recv_sem=rsems.at[sid0],
                    device_id=(d2,), device_id_type=pl.DeviceIdType.MESH)
                r.start()
                return (r,)

            def finish(st):
                st[0].wait()
                z_ref[pl.ds(off, 64), :] += comm[...]

            return issue, finish

        def bfly_ag_round(axis, dist, qb):
            _, b0, b1, d1, d2 = bfly[axis]
            if dist == 2:
                partner, nrows = d2, 32
                off = qb + b0 * 64 + b1 * 32
            else:
                partner, nrows = d1, 64
                off = qb + b0 * 64
            sid0 = next_sid2()

            def issue():
                r = pltpu.make_async_remote_copy(
                    src_ref=out_ref.at[pl.ds(off, nrows), :],
                    dst_ref=out_ref.at[pl.ds(off, nrows), :],
                    send_sem=ssems.at[sid0], recv_sem=rsems.at[sid0],
                    device_id=(partner,), device_id_type=pl.DeviceIdType.MESH)
                r.start()
                return (r,)

            def finish(st):
                st[0].wait()

            return issue, finish

        def build_half(row0, ax1, ax2, comm_big, slot_b0, comm_m1, comm_m2):
            chunk = 128
            x1 = ring[ax1][0]

            def base1(c):
                return row0 + c * chunk

            qb = row0 + ((x1 + 1) % 4) * chunk

            rounds = []
            for s in range(3):
                rounds.append(rs_round(s, ax1, base1, chunk // 2,
                                       comm_big, slot_b0 + 2 * s, z_ref))
            rounds.append(bfly_rs_round(ax2, 1, qb, comm_m1))
            rounds.append(bfly_x2_round(ax2, qb, comm_m2))
            rounds.append(bfly_ag_round(ax2, 1, qb))
            for s in range(3):
                rounds.append(ag_round(s, ax1, base1, chunk // 2, out_ref))
            _, b0, _, _, _ = bfly[ax2]
            silu_rows = qb + b0 * 64
            return rounds, silu_rows, 64

        b16_ref[...] = b_ref[...].astype(jnp.bfloat16)

        def mm_block(c, row0, rows):
            b = row0 + c * rows
            z_ref[pl.ds(b, rows), :] = jnp.dot(
                a_ref[pl.ds(b, rows), :].astype(jnp.bfloat16), b16_ref[...],
                preferred_element_type=jnp.float32).astype(jnp.bfloat16)

        PROBE_MM_ONLY = False
        ra, silu_a, sub_a = build_half(0, "P", "Z", comm_pa, 0, m1a, m2a)
        rb, silu_b, sub_b = build_half(512, "Z", "P", comm_zb, 0, m1b, m2b)
        if PROBE_MM_ONLY:
            for c in range(4):
                mm_block(c, 0, 128)
                mm_block(c, 512, 128)
            zc = z_ref[...].astype(jnp.float32)
            out_ref[...] = (zc * (1.0 / (1.0 + jnp.exp(-zc)))).astype(jnp.bfloat16)
            return

        def mm64(row):
            z_ref[pl.ds(row, 64), :] = jnp.dot(
                a_ref[pl.ds(row, 64), :].astype(jnp.bfloat16), b16_ref[...],
                preferred_element_type=jnp.float32).astype(jnp.bfloat16)

        mm64(q * 128)
        mm64(((q + 2) % 4) * 128 + 64)
        mm64(512 + pos_z * 128)
        mm64(512 + ((pos_z + 2) % 4) * 128 + 64)
        sta = ra[0][0]()
        stb = rb[0][0]()
        for c, off in (((q + 3) % 4, 0), ((q + 3) % 4, 64)):
            mm64(c * 128 + off)
        for c, off in (((pos_z + 3) % 4, 0), ((pos_z + 3) % 4, 64)):
            mm64(512 + c * 128 + off)
        mm64(((q + 2) % 4) * 128)
        mm64(q * 128 + 64)
        mm64(512 + ((pos_z + 2) % 4) * 128)
        mm64(512 + pos_z * 128 + 64)
        mm64(((q + 1) % 4) * 128)
        mm64(((q + 1) % 4) * 128 + 64)
        mm64(512 + ((pos_z + 1) % 4) * 128)
        mm64(512 + ((pos_z + 1) % 4) * 128 + 64)
        ra[0][1](sta)
        rb[0][1](stb)

        for i in range(1, 9):
            sta = ra[i][0]()
            stb = rb[i][0]()
            ra[i][1](sta)
            rb[i][1](stb)
            if i == 4:
                for rows0, sb in ((silu_a, sub_a), (silu_b, sub_b)):
                    zc = z_ref[pl.ds(rows0, sb), :].astype(jnp.float32)
                    out_ref[pl.ds(rows0, sb), :] = (
                        zc * (1.0 / (1.0 + jnp.exp(-zc)))
                    ).astype(jnp.bfloat16)

    return pl.pallas_call(
        body,
        out_shape=jax.ShapeDtypeStruct((M, N), jnp.bfloat16),
        in_specs=[pl.BlockSpec(memory_space=pltpu.VMEM),
                  pl.BlockSpec(memory_space=pltpu.VMEM)],
        out_specs=pl.BlockSpec(memory_space=pltpu.VMEM),
        scratch_shapes=[
            pltpu.VMEM((512, N), jnp.bfloat16),
            pltpu.VMEM((M, N), jnp.bfloat16),
            pltpu.VMEM((6, 64, N), jnp.bfloat16),
            pltpu.VMEM((6, 64, N), jnp.bfloat16),
            pltpu.VMEM((64, N), jnp.bfloat16),
            pltpu.VMEM((64, N), jnp.bfloat16),
            pltpu.VMEM((64, N), jnp.bfloat16),
            pltpu.VMEM((64, N), jnp.bfloat16),
            pltpu.SemaphoreType.DMA((64,)),
            pltpu.SemaphoreType.DMA((64,)),
        ],
        compiler_params=pltpu.CompilerParams(collective_id=0),
    )(A, B)
